# baseline (speedup 1.0000x reference)
import functools
import math

import jax
import jax.numpy as jnp
from jax.experimental import pallas as pl
from jax.experimental.pallas import tpu as pltpu
from jax.experimental.pallas import tpu_sc as plsc

N = 10000
D = 128
K = 27
KG = K - 1
NPAD = 10240
SLICES = (7168, 3072)
NW = 32
CH = 416
NZ = 1024


def _prep_body(emb27_ref, sh27_ref, weight_ref, wsc_ref, o_ref):
    w = jnp.dot(emb27_ref[...], weight_ref[...],
                preferred_element_type=jnp.float32)
    scale = 1.0 / (K * math.sqrt(float(D)))
    o_ref[...] = w * sh27_ref[...] * scale
    o_ref[13:14, :] = o_ref[13:14, :] + wsc_ref[...] * (1.0 / math.sqrt(float(D)))


def _prep(emb27, sh27, weight, wsc_row):
    return pl.pallas_call(
        _prep_body,
        out_shape=jax.ShapeDtypeStruct((K, D * D), jnp.float32),
    )(emb27, sh27, weight, wsc_row)


def _sc_gather(x_pad, idx_slice, ns):
    total_s = ns * KG
    bpw = total_s // NW
    nchunk = bpw // CH
    assert bpw % CH == 0 and total_s % NW == 0
    mesh = plsc.VectorSubcoreMesh(core_axis_name="c", subcore_axis_name="s")
    out_type = jax.ShapeDtypeStruct((total_s, D), x_pad.dtype)

    @functools.partial(
        pl.kernel, out_type=out_type, mesh=mesh,
        scratch_types=[
            pltpu.VMEM((CH,), jnp.int32),
            pltpu.VMEM((CH,), jnp.int32),
            pltpu.VMEM((CH, D), jnp.float32),
            pltpu.VMEM((CH, D), jnp.float32),
            pltpu.SemaphoreType.DMA,
            pltpu.SemaphoreType.DMA,
            pltpu.SemaphoreType.DMA,
            pltpu.SemaphoreType.DMA,
        ],
    )
    def gather_kernel(x_hbm, i_hbm, o_hbm, idx0, idx1, rows0, rows1,
                      gsem0, gsem1, wsem0, wsem1):
        wid = jax.lax.axis_index("s") * 2 + jax.lax.axis_index("c")
        base_w = wid * bpw
        idx_v = (idx0, idx1)
        rows_v = (rows0, rows1)
        gsem = (gsem0, gsem1)
        wsem = (wsem0, wsem1)

        gcopy = [None] * nchunk
        wcopy = [None] * nchunk
        for c in range(nchunk):
            b = c % 2
            if c >= 2:
                wcopy[c - 2].wait()
            base = base_w + c * CH
            pltpu.sync_copy(i_hbm.at[pl.ds(base, CH)], idx_v[b])
            gcopy[c] = pltpu.async_copy(x_hbm.at[idx_v[b]], rows_v[b], gsem[b])
            if c >= 1:
                gcopy[c - 1].wait()
                pb = (c - 1) % 2
                wcopy[c - 1] = pltpu.async_copy(
                    rows_v[pb], o_hbm.at[pl.ds(base_w + (c - 1) * CH, CH)],
                    wsem[pb])
        gcopy[nchunk - 1].wait()
        lb = (nchunk - 1) % 2
        wcopy[nchunk - 1] = pltpu.async_copy(
            rows_v[lb], o_hbm.at[pl.ds(base_w + (nchunk - 1) * CH, CH)],
            wsem[lb])
        wcopy[nchunk - 2].wait()
        wcopy[nchunk - 1].wait()

    return gather_kernel(x_pad, idx_slice)


def _conv_body(g_ref, a_ref, xs_ref, o_ref):
    k = pl.program_id(0)

    @pl.when(k == 0)
    def _():
        o_ref[...] = jnp.zeros_like(o_ref)

    @pl.when(k != 13)
    def _():
        o_ref[...] += jnp.dot(g_ref[...], a_ref[k],
                              preferred_element_type=jnp.float32)

    @pl.when(k == 13)
    def _():
        o_ref[...] += jnp.dot(xs_ref[...], a_ref[13],
                              preferred_element_type=jnp.float32)


def _gmap(k):
    return jnp.where(k > 13, k - 1, jnp.minimum(k, 12))


def _conv(G, Akdd, xs, ns):
    return pl.pallas_call(
        _conv_body,
        grid=(K,),
        in_specs=[
            pl.BlockSpec((ns, D), lambda k: (_gmap(k), 0)),
            pl.BlockSpec((K, D, D), lambda k: (0, 0, 0)),
            pl.BlockSpec((ns, D), lambda k: (0, 0)),
        ],
        out_specs=pl.BlockSpec((ns, D), lambda k: (0, 0)),
        out_shape=jax.ShapeDtypeStruct((ns, D), jnp.float32),
    )(G, Akdd, xs)


def kernel(x, W_sc, weight, emb, sh, neigh_idx):
    x = x.astype(jnp.float32)
    x_pad = jnp.concatenate([x, jnp.zeros((NZ, D), x.dtype)], axis=0)
    idx = neigh_idx.astype(jnp.int32)
    idxT = idx.T
    idx_cols = jnp.pad(jnp.concatenate([idxT[:13], idxT[14:]], axis=0),
                       ((0, 0), (0, NPAD - N)), constant_values=N)
    spread = N + jax.lax.rem(
        jax.lax.broadcasted_iota(jnp.int32, (KG, NPAD), 1), jnp.int32(NZ))
    idx_cols = jnp.where(idx_cols == N, spread, idx_cols)
    emb27 = emb.transpose(2, 1, 0, 3).reshape(K, -1)
    sh27 = sh[..., 0].transpose(2, 1, 0).reshape(K, 1)
    wsc_row = W_sc.reshape(1, D * D)

    Akdd = _prep(emb27, sh27, weight, wsc_row).reshape(K, D, D)
    outs = []
    n0 = 0
    for ns in SLICES:
        idx_s = jax.lax.slice(idx_cols, (0, n0), (KG, n0 + ns)).reshape(KG * ns)
        xs = jax.lax.slice(x_pad, (n0, 0), (n0 + ns, D))
        G = _sc_gather(x_pad, idx_s, ns)
        outs.append(_conv(G, Akdd, xs, ns))
        n0 += ns
    return jnp.concatenate(outs, axis=0)[:N]


# speedup vs baseline: 1.1079x; 1.1079x over previous
import functools
import math

import jax
import jax.numpy as jnp
from jax.experimental import pallas as pl
from jax.experimental.pallas import tpu as pltpu
from jax.experimental.pallas import tpu_sc as plsc

N = 10000
D = 128
K = 27
NPAD = 10240
SLICES = (7168, 3072)
NW = 32
CH = 432
NZ = 1024


def _prep_body(emb27_ref, sh27_ref, weight_ref, wsc_ref, o_ref):
    w = jnp.dot(emb27_ref[...], weight_ref[...],
                preferred_element_type=jnp.float32)
    scale = 1.0 / (K * math.sqrt(float(D)))
    o_ref[...] = w * sh27_ref[...] * scale
    o_ref[13:14, :] = o_ref[13:14, :] + wsc_ref[...] * (1.0 / math.sqrt(float(D)))


def _prep(emb27, sh27, weight, wsc_row):
    return pl.pallas_call(
        _prep_body,
        out_shape=jax.ShapeDtypeStruct((K, D * D), jnp.float32),
    )(emb27, sh27, weight, wsc_row)


def _sc_gather(x_pad, idx_slice, ns):
    total_s = ns * K
    bpw = total_s // NW
    nchunk = bpw // CH
    assert bpw % CH == 0 and total_s % NW == 0
    mesh = plsc.VectorSubcoreMesh(core_axis_name="c", subcore_axis_name="s")
    out_type = jax.ShapeDtypeStruct((total_s, D), x_pad.dtype)

    @functools.partial(
        pl.kernel, out_type=out_type, mesh=mesh,
        scratch_types=[
            pltpu.VMEM((CH,), jnp.int32),
            pltpu.VMEM((CH,), jnp.int32),
            pltpu.VMEM((CH, D), jnp.float32),
            pltpu.VMEM((CH, D), jnp.float32),
            pltpu.SemaphoreType.DMA,
            pltpu.SemaphoreType.DMA,
            pltpu.SemaphoreType.DMA,
            pltpu.SemaphoreType.DMA,
        ],
    )
    def gather_kernel(x_hbm, i_hbm, o_hbm, idx0, idx1, rows0, rows1,
                      gsem0, gsem1, wsem0, wsem1):
        wid = jax.lax.axis_index("s") * 2 + jax.lax.axis_index("c")
        base_w = wid * bpw
        idx_v = (idx0, idx1)
        rows_v = (rows0, rows1)
        gsem = (gsem0, gsem1)
        wsem = (wsem0, wsem1)

        gcopy = [None] * nchunk
        wcopy = [None] * nchunk
        for c in range(nchunk):
            b = c % 2
            if c >= 2:
                wcopy[c - 2].wait()
            base = base_w + c * CH
            pltpu.sync_copy(i_hbm.at[pl.ds(base, CH)], idx_v[b])
            gcopy[c] = pltpu.async_copy(x_hbm.at[idx_v[b]], rows_v[b], gsem[b])
            if c >= 1:
                gcopy[c - 1].wait()
                pb = (c - 1) % 2
                wcopy[c - 1] = pltpu.async_copy(
                    rows_v[pb], o_hbm.at[pl.ds(base_w + (c - 1) * CH, CH)],
                    wsem[pb])
        gcopy[nchunk - 1].wait()
        lb = (nchunk - 1) % 2
        wcopy[nchunk - 1] = pltpu.async_copy(
            rows_v[lb], o_hbm.at[pl.ds(base_w + (nchunk - 1) * CH, CH)],
            wsem[lb])
        wcopy[nchunk - 2].wait()
        wcopy[nchunk - 1].wait()

    return gather_kernel(x_pad, idx_slice)


def _conv_body(g_ref, a_ref, o_ref):
    k = pl.program_id(0)

    @pl.when(k == 0)
    def _():
        o_ref[...] = jnp.zeros_like(o_ref)

    o_ref[...] += jnp.dot(g_ref[...], a_ref[k],
                          preferred_element_type=jnp.float32)


def _conv(G, Akdd, ns):
    return pl.pallas_call(
        _conv_body,
        grid=(K,),
        in_specs=[
            pl.BlockSpec((ns, D), lambda k: (k, 0)),
            pl.BlockSpec((K, D, D), lambda k: (0, 0, 0)),
        ],
        out_specs=pl.BlockSpec((ns, D), lambda k: (0, 0)),
        out_shape=jax.ShapeDtypeStruct((ns, D), jnp.float32),
    )(G, Akdd)


def kernel(x, W_sc, weight, emb, sh, neigh_idx):
    x = x.astype(jnp.float32)
    x_pad = jnp.concatenate([x, jnp.zeros((NZ, D), x.dtype)], axis=0)
    idx = neigh_idx.astype(jnp.int32)
    idx_cols = jnp.pad(idx.T, ((0, 0), (0, NPAD - N)), constant_values=N)
    spread = N + jax.lax.rem(
        jax.lax.broadcasted_iota(jnp.int32, (K, NPAD), 1), jnp.int32(NZ))
    idx_cols = jnp.where(idx_cols == N, spread, idx_cols)
    emb27 = emb.transpose(2, 1, 0, 3).reshape(K, -1)
    sh27 = sh[..., 0].transpose(2, 1, 0).reshape(K, 1)
    wsc_row = W_sc.reshape(1, D * D)

    Akdd = _prep(emb27, sh27, weight, wsc_row).reshape(K, D, D)
    outs = []
    n0 = 0
    for ns in SLICES:
        idx_s = jax.lax.slice(idx_cols, (0, n0), (K, n0 + ns)).reshape(K * ns)
        G = _sc_gather(x_pad, idx_s, ns)
        outs.append(_conv(G, Akdd, ns))
        n0 += ns
    return jnp.concatenate(outs, axis=0)[:N]
